# baseline (speedup 1.0000x reference)
import functools

import jax
import jax.numpy as jnp
from jax import lax
from jax.experimental import pallas as pl
from jax.experimental.pallas import tpu as pltpu
from jax.experimental.pallas import tpu_sc as plsc

_N = 10000
_E = 320000
_D = 128
_G = 64

_NC = 2
_NS = 16
_NW = _NC * _NS
_EPW = _E // _NW
_CH = 80
_NK = 4
_NCHUNK = 128
_EPW2 = _NCHUNK * _CH
_NACC = _N + 8
_RPT = 624
_REM = _N - _NS * _RPT


def _make_agg_kernel():
    mesh = plsc.VectorSubcoreMesh(core_axis_name="c", subcore_axis_name="s")

    @functools.partial(
        pl.kernel,
        mesh=mesh,
        out_type=jax.ShapeDtypeStruct((_NC * _N, _D), jnp.float32),
        scratch_types=(
            [pltpu.VMEM((_CH,), jnp.int32)] * _NK
            + [pltpu.VMEM((_CH,), jnp.int32)] * _NK
            + [pltpu.VMEM((_CH, _D), jnp.float32)] * _NK
            + [pltpu.VMEM_SHARED((_NACC, _D), jnp.float32)]
            + [pltpu.SemaphoreType.DMA] * (4 * _NK)
        ),
    )
    def agg(h_hbm, srcf_hbm, dstf_hbm, zeros_hbm, out_hbm, *rest):
        sbufs = rest[:_NK]
        dbufs = rest[_NK:2 * _NK]
        rows = rest[2 * _NK:3 * _NK]
        accum = rest[3 * _NK]
        semi = rest[3 * _NK + 1:3 * _NK + 1 + _NK]
        semd = rest[3 * _NK + 1 + _NK:3 * _NK + 1 + 2 * _NK]
        semg = rest[3 * _NK + 1 + 2 * _NK:3 * _NK + 1 + 3 * _NK]
        sems = rest[3 * _NK + 1 + 3 * _NK:]
        c = lax.axis_index("c")
        s = lax.axis_index("s")
        pltpu.sync_copy(zeros_hbm.at[pl.ds(s * _RPT, _RPT)],
                        accum.at[pl.ds(s * _RPT, _RPT)])

        @pl.when(s == 0)
        def _():
            pltpu.sync_copy(zeros_hbm.at[pl.ds(_NS * _RPT, _REM)],
                            accum.at[pl.ds(_NS * _RPT, _REM)])

        plsc.subcore_barrier()

        wid = c * _NS + s
        ebase = wid * _EPW2

        def fire_idx(i, p):
            pltpu.async_copy(srcf_hbm.at[pl.ds(ebase + i * _CH, _CH)],
                             sbufs[p], semi[p])
            pltpu.async_copy(dstf_hbm.at[pl.ds(ebase + i * _CH, _CH)],
                             dbufs[p], semd[p])

        def fire_gather(p):
            pltpu.make_async_copy(srcf_hbm.at[pl.ds(0, _CH)],
                                  sbufs[p], semi[p]).wait()
            pltpu.async_copy(h_hbm.at[sbufs[p]], rows[p], semg[p])

        def scatter(p):
            pltpu.make_async_copy(dstf_hbm.at[pl.ds(0, _CH)],
                                  dbufs[p], semd[p]).wait()
            pltpu.make_async_copy(h_hbm.at[pl.ds(0, _CH)],
                                  rows[p], semg[p]).wait()
            pltpu.async_copy(rows[p], accum.at[dbufs[p]], sems[p], add=True)

        def wait_scatter(p):
            pltpu.make_async_copy(rows[p], accum.at[pl.ds(0, _CH)],
                                  sems[p]).wait()

        def block(j, carry):
            for p in range(_NK):
                @pl.when(j > 0)
                def _():
                    wait_scatter(p)
                fire_idx(j * _NK + p, p)
            for p in range(_NK):
                fire_gather(p)
            for p in range(_NK):
                scatter(p)
            return carry

        lax.fori_loop(0, _NCHUNK // _NK, block, 0)
        for p in range(_NK):
            wait_scatter(p)
        plsc.subcore_barrier()
        pltpu.sync_copy(accum.at[pl.ds(s * _RPT, _RPT)],
                        out_hbm.at[pl.ds(c * _N + s * _RPT, _RPT)])

        @pl.when(s == 0)
        def _():
            pltpu.sync_copy(accum.at[pl.ds(_NS * _RPT, _REM)],
                            out_hbm.at[pl.ds(c * _N + _NS * _RPT, _REM)])

    return agg


def _mlp(h, parts, W1, b1, W2, b2, relu_out):
    BN = 2000
    nblk = _N // BN

    def body(h_ref, a0_ref, a1_ref, W1_ref, b1_ref, W2_ref, b2_ref, o_ref):
        t = h_ref[...] + a0_ref[...] + a1_ref[...]
        t = jnp.dot(t, W1_ref[...], preferred_element_type=jnp.float32,
                    precision=lax.Precision.HIGHEST) + b1_ref[...]
        t = jnp.maximum(t, 0.0)
        t = jnp.dot(t, W2_ref[...], preferred_element_type=jnp.float32,
                    precision=lax.Precision.HIGHEST) + b2_ref[...]
        if relu_out:
            t = jnp.maximum(t, 0.0)
        o_ref[...] = t

    return pl.pallas_call(
        body,
        grid=(nblk,),
        in_specs=[
            pl.BlockSpec((BN, _D), lambda i: (i, 0)),
            pl.BlockSpec((BN, _D), lambda i: (i, 0)),
            pl.BlockSpec((BN, _D), lambda i: (i + nblk, 0)),
            pl.BlockSpec((_D, _D), lambda i: (0, 0)),
            pl.BlockSpec((1, _D), lambda i: (0, 0)),
            pl.BlockSpec((_D, _D), lambda i: (0, 0)),
            pl.BlockSpec((1, _D), lambda i: (0, 0)),
        ],
        out_specs=pl.BlockSpec((BN, _D), lambda i: (i, 0)),
        out_shape=jax.ShapeDtypeStruct((_N, _D), jnp.float32),
    )(h, parts, parts, W1, b1.reshape(1, _D), W2, b2.reshape(1, _D))


def _pool(h, batch2d, lin_W, lin_b):
    def body(h_ref, b_ref, W_ref, bias_ref, o_ref):
        seg = b_ref[...]
        gids = lax.broadcasted_iota(jnp.int32, (_G, _N), 0)
        mask = (seg == gids).astype(jnp.float32)
        psum = jnp.dot(mask, h_ref[...], preferred_element_type=jnp.float32,
                       precision=lax.Precision.HIGHEST)
        cnt = jnp.sum(mask, axis=1, keepdims=True)
        pooled = psum / jnp.maximum(cnt, 1.0)
        o_ref[...] = jnp.dot(pooled, W_ref[...],
                             preferred_element_type=jnp.float32,
                             precision=lax.Precision.HIGHEST) + bias_ref[...]

    return pl.pallas_call(
        body,
        out_shape=jax.ShapeDtypeStruct((_G, _D), jnp.float32),
    )(h, batch2d, lin_W, lin_b.reshape(1, _D))


def kernel(x, edge_index, batch, gin0_W1, gin0_b1, gin0_W2, gin0_b2,
           gin1_W1, gin1_b1, gin1_W2, gin1_b2,
           gin2_W1, gin2_b1, gin2_W2, gin2_b2, lin_W, lin_b):
    pad = _EPW2 - _EPW
    src = jnp.concatenate(
        [edge_index[0].reshape(_NW, _EPW),
         jnp.zeros((_NW, pad), jnp.int32)], axis=1).reshape(-1)
    dst = jnp.concatenate(
        [edge_index[1].reshape(_NW, _EPW),
         jnp.full((_NW, pad), _N, jnp.int32)], axis=1).reshape(-1)
    zeros = jnp.zeros((_N, _D), jnp.float32)
    aggk = _make_agg_kernel()

    def layer(h, W1, b1, W2, b2, relu_out):
        parts = aggk(h, src, dst, zeros)
        return _mlp(h, parts, W1, b1, W2, b2, relu_out)

    h = layer(x, gin0_W1, gin0_b1, gin0_W2, gin0_b2, True)
    h = layer(h, gin1_W1, gin1_b1, gin1_W2, gin1_b2, True)
    h = layer(h, gin2_W1, gin2_b1, gin2_W2, gin2_b2, False)
    return _pool(h, batch.reshape(1, _N), lin_W, lin_b)


# speedup vs baseline: 1.3343x; 1.3343x over previous
import functools

import jax
import jax.numpy as jnp
from jax import lax
from jax.experimental import pallas as pl
from jax.experimental.pallas import tpu as pltpu
from jax.experimental.pallas import tpu_sc as plsc

_N = 10000
_E = 320000
_D = 128
_G = 64

_NC = 2
_NS = 16
_NW = _NC * _NS
_EPW = _E // _NW
_CH = 80
_NCHUNK = _EPW // _CH
_RPT = 624
_REM = _N - _NS * _RPT


def _make_agg_kernel():
    mesh = plsc.VectorSubcoreMesh(core_axis_name="c", subcore_axis_name="s")

    @functools.partial(
        pl.kernel,
        mesh=mesh,
        out_type=jax.ShapeDtypeStruct((_NC * _N, _D), jnp.float32),
        scratch_types=[
            pltpu.VMEM((_CH,), jnp.int32),
            pltpu.VMEM((_CH,), jnp.int32),
            pltpu.VMEM((_CH, _D), jnp.float32),
            pltpu.VMEM_SHARED((_N, _D), jnp.float32),
            pltpu.SemaphoreType.DMA,
        ],
    )
    def agg(h_hbm, src_hbm, dst_hbm, zeros_hbm, out_hbm,
            idx_s, idx_d, rows, accum, sem):
        c = lax.axis_index("c")
        s = lax.axis_index("s")
        pltpu.sync_copy(zeros_hbm.at[pl.ds(s * _RPT, _RPT)],
                        accum.at[pl.ds(s * _RPT, _RPT)])

        @pl.when(s == 0)
        def _():
            pltpu.sync_copy(zeros_hbm.at[pl.ds(_NS * _RPT, _REM)],
                            accum.at[pl.ds(_NS * _RPT, _REM)])

        plsc.subcore_barrier()

        base = (c * _NS + s) * _EPW

        def body(i, carry):
            off = base + i * _CH
            pltpu.sync_copy(src_hbm.at[pl.ds(off, _CH)], idx_s)
            pltpu.sync_copy(dst_hbm.at[pl.ds(off, _CH)], idx_d)
            pltpu.async_copy(h_hbm.at[idx_s], rows, sem).wait()
            pltpu.sync_copy(rows, accum.at[idx_d], add=True)
            return carry

        lax.fori_loop(0, _NCHUNK, body, 0)
        plsc.subcore_barrier()
        pltpu.sync_copy(accum.at[pl.ds(s * _RPT, _RPT)],
                        out_hbm.at[pl.ds(c * _N + s * _RPT, _RPT)])

        @pl.when(s == 0)
        def _():
            pltpu.sync_copy(accum.at[pl.ds(_NS * _RPT, _REM)],
                            out_hbm.at[pl.ds(c * _N + _NS * _RPT, _REM)])

    return agg


def _mlp(h, parts, W1, b1, W2, b2, relu_out):
    BN = 2000
    nblk = _N // BN

    def body(h_ref, a0_ref, a1_ref, W1_ref, b1_ref, W2_ref, b2_ref, o_ref):
        t = h_ref[...] + a0_ref[...] + a1_ref[...]
        t = jnp.dot(t, W1_ref[...], preferred_element_type=jnp.float32,
                    precision=lax.Precision.HIGHEST) + b1_ref[...]
        t = jnp.maximum(t, 0.0)
        t = jnp.dot(t, W2_ref[...], preferred_element_type=jnp.float32,
                    precision=lax.Precision.HIGHEST) + b2_ref[...]
        if relu_out:
            t = jnp.maximum(t, 0.0)
        o_ref[...] = t

    return pl.pallas_call(
        body,
        grid=(nblk,),
        in_specs=[
            pl.BlockSpec((BN, _D), lambda i: (i, 0)),
            pl.BlockSpec((BN, _D), lambda i: (i, 0)),
            pl.BlockSpec((BN, _D), lambda i: (i + nblk, 0)),
            pl.BlockSpec((_D, _D), lambda i: (0, 0)),
            pl.BlockSpec((1, _D), lambda i: (0, 0)),
            pl.BlockSpec((_D, _D), lambda i: (0, 0)),
            pl.BlockSpec((1, _D), lambda i: (0, 0)),
        ],
        out_specs=pl.BlockSpec((BN, _D), lambda i: (i, 0)),
        out_shape=jax.ShapeDtypeStruct((_N, _D), jnp.float32),
    )(h, parts, parts, W1, b1.reshape(1, _D), W2, b2.reshape(1, _D))


def _mlp_pool(h, parts, W1, b1, W2, b2, batch_col, lin_W, lin_b):
    BN = 2000
    nblk = _N // BN

    def body(h_ref, a0_ref, a1_ref, W1_ref, b1_ref, W2_ref, b2_ref,
             bat_ref, lW_ref, lb_ref, o_ref, acc_ref, cnt_ref):
        i = pl.program_id(0)
        t = h_ref[...] + a0_ref[...] + a1_ref[...]
        t = jnp.dot(t, W1_ref[...], preferred_element_type=jnp.float32,
                    precision=lax.Precision.HIGHEST) + b1_ref[...]
        t = jnp.maximum(t, 0.0)
        t = jnp.dot(t, W2_ref[...], preferred_element_type=jnp.float32,
                    precision=lax.Precision.HIGHEST) + b2_ref[...]
        gids = lax.broadcasted_iota(jnp.int32, (BN, _G), 1)
        mask = (bat_ref[...] == gids).astype(jnp.float32)
        psum = lax.dot_general(mask, t, (((0,), (0,)), ((), ())),
                               preferred_element_type=jnp.float32,
                               precision=lax.Precision.HIGHEST)
        cnt = jnp.broadcast_to(jnp.sum(mask, axis=0).reshape(_G, 1), (_G, _D))

        @pl.when(i == 0)
        def _():
            acc_ref[...] = psum
            cnt_ref[...] = cnt

        @pl.when(i > 0)
        def _():
            acc_ref[...] += psum
            cnt_ref[...] += cnt

        @pl.when(i == nblk - 1)
        def _():
            pooled = acc_ref[...] / jnp.maximum(cnt_ref[...], 1.0)
            o_ref[...] = jnp.dot(pooled, lW_ref[...],
                                 preferred_element_type=jnp.float32,
                                 precision=lax.Precision.HIGHEST) + lb_ref[...]

    return pl.pallas_call(
        body,
        grid=(nblk,),
        in_specs=[
            pl.BlockSpec((BN, _D), lambda i: (i, 0)),
            pl.BlockSpec((BN, _D), lambda i: (i, 0)),
            pl.BlockSpec((BN, _D), lambda i: (i + nblk, 0)),
            pl.BlockSpec((_D, _D), lambda i: (0, 0)),
            pl.BlockSpec((1, _D), lambda i: (0, 0)),
            pl.BlockSpec((_D, _D), lambda i: (0, 0)),
            pl.BlockSpec((1, _D), lambda i: (0, 0)),
            pl.BlockSpec((BN, 1), lambda i: (i, 0)),
            pl.BlockSpec((_D, _D), lambda i: (0, 0)),
            pl.BlockSpec((1, _D), lambda i: (0, 0)),
        ],
        out_specs=pl.BlockSpec((_G, _D), lambda i: (0, 0)),
        out_shape=jax.ShapeDtypeStruct((_G, _D), jnp.float32),
        scratch_shapes=[pltpu.VMEM((_G, _D), jnp.float32),
                        pltpu.VMEM((_G, _D), jnp.float32)],
    )(h, parts, parts, W1, b1.reshape(1, _D), W2, b2.reshape(1, _D),
      batch_col, lin_W, lin_b.reshape(1, _D))


def kernel(x, edge_index, batch, gin0_W1, gin0_b1, gin0_W2, gin0_b2,
           gin1_W1, gin1_b1, gin1_W2, gin1_b2,
           gin2_W1, gin2_b1, gin2_W2, gin2_b2, lin_W, lin_b):
    src = edge_index[0]
    dst = edge_index[1]
    zeros = jnp.zeros((_N, _D), jnp.float32)
    aggk = _make_agg_kernel()

    def layer(h, W1, b1, W2, b2, relu_out):
        parts = aggk(h, src, dst, zeros)
        return _mlp(h, parts, W1, b1, W2, b2, relu_out)

    h = layer(x, gin0_W1, gin0_b1, gin0_W2, gin0_b2, True)
    h = layer(h, gin1_W1, gin1_b1, gin1_W2, gin1_b2, True)
    parts = aggk(h, src, dst, zeros)
    return _mlp_pool(h, parts, gin2_W1, gin2_b1, gin2_W2, gin2_b2,
                     batch.reshape(_N, 1), lin_W, lin_b)
